# baseline (speedup 1.0000x reference)
import functools

import jax
import jax.numpy as jnp
from jax import lax
from jax.experimental import pallas as pl
from jax.experimental.pallas import tpu as pltpu
from jax.experimental.pallas import tpu_sc as plsc

N = 10000
E = 320000
D = 128
H = 128
G = 64
EPS = 1e-5

NC = 2
NS = 16
NW = NC * NS
CH = 128
GSZ = 40
K = -(-E // (NW * CH * GSZ)) * GSZ
NG = K // GSZ
E_PAD = NW * K * CH
NP = 10112
RPT = NP // NS


@functools.cache
def _sc_kernels():
    mesh = plsc.VectorSubcoreMesh(core_axis_name="c", subcore_axis_name="s")
    deg = functools.partial(
        pl.kernel,
        out_type=jax.ShapeDtypeStruct((NC * NP,), jnp.float32),
        mesh=mesh,
        scratch_types=[
            pltpu.VMEM((K, CH), jnp.int32),
            pltpu.VMEM((CH,), jnp.float32),
            pltpu.VMEM((RPT,), jnp.float32),
            pltpu.VMEM_SHARED((NP,), jnp.float32),
        ],
    )(_sc_degree_body)
    edge = functools.partial(
        pl.kernel,
        out_type=jax.ShapeDtypeStruct((NC, NP, H), jnp.float32),
        mesh=mesh,
        scratch_types=[
            pltpu.VMEM((GSZ, CH), jnp.int32),
            pltpu.VMEM((GSZ, CH), jnp.int32),
            pltpu.VMEM((CH, H), jnp.float32),
            pltpu.VMEM((CH, H), jnp.float32),
            pltpu.VMEM_SHARED((NP, H), jnp.float32),
            pltpu.SemaphoreType.DMA,
            pltpu.SemaphoreType.DMA,
            pltpu.SemaphoreType.DMA,
            pltpu.SemaphoreType.DMA,
        ],
    )(_sc_edge_pass_body)
    return deg, edge


def _sc_degree_body(dst_hbm, out_hbm, dst_v, ones_v, stage_v, deg_sh):
    c = lax.axis_index("c")
    s = lax.axis_index("s")
    wid = c * NS + s

    @pl.loop(0, CH, step=16)
    def _(i):
        ones_v[pl.ds(i, 16)] = jnp.zeros((16,), jnp.float32)

    @pl.loop(0, RPT - (RPT % CH), step=CH)
    def _(r):
        pltpu.sync_copy(ones_v, deg_sh.at[pl.ds(s * RPT + r, CH)])

    rem = RPT % CH
    if rem:
        pltpu.sync_copy(ones_v.at[pl.ds(0, rem)],
                        deg_sh.at[pl.ds(s * RPT + (RPT - rem), rem)])

    @pl.loop(0, CH, step=16)
    def _(i):
        ones_v[pl.ds(i, 16)] = jnp.ones((16,), jnp.float32)

    pltpu.sync_copy(dst_hbm.at[wid], dst_v)
    plsc.subcore_barrier()

    @pl.loop(0, K)
    def _(j):
        pltpu.sync_copy(ones_v, deg_sh.at[dst_v.at[j]], add=True)

    plsc.subcore_barrier()
    pltpu.sync_copy(deg_sh.at[pl.ds(s * RPT, RPT)], stage_v)
    pltpu.sync_copy(stage_v, out_hbm.at[pl.ds(c * NP + s * RPT, RPT)])


def _sc_edge_pass_body(u_hbm, src_hbm, dst_hbm, out_hbm,
                       src_v, dst_v, rows_a, rows_b, acc_sh,
                       gsa, gsb, ssa, ssb):
    c = lax.axis_index("c")
    s = lax.axis_index("s")
    wid = c * NS + s

    @pl.loop(0, CH)
    def _(i):
        @pl.loop(0, H, step=16)
        def _(j):
            rows_a[i, pl.ds(j, 16)] = jnp.zeros((16,), jnp.float32)

    @pl.loop(0, RPT - (RPT % CH), step=CH)
    def _(r):
        pltpu.sync_copy(rows_a, acc_sh.at[pl.ds(s * RPT + r, CH)])

    rem = RPT % CH
    if rem:
        pltpu.sync_copy(rows_a.at[pl.ds(0, rem)],
                        acc_sh.at[pl.ds(s * RPT + (RPT - rem), rem)])

    plsc.subcore_barrier()

    @pl.loop(0, NG)
    def _(g):
        pltpu.sync_copy(src_hbm.at[wid, pl.ds(g * GSZ, GSZ)], src_v)
        pltpu.sync_copy(dst_hbm.at[wid, pl.ds(g * GSZ, GSZ)], dst_v)

        pltpu.async_copy(u_hbm.at[src_v.at[0]], rows_a, gsa)
        pltpu.async_copy(u_hbm.at[src_v.at[1]], rows_b, gsb)

        @pl.loop(0, GSZ - 2, step=2)
        def _(j):
            pltpu.make_async_copy(u_hbm.at[src_v.at[j]], rows_a, gsa).wait()
            pltpu.async_copy(rows_a, acc_sh.at[dst_v.at[j]], ssa, add=True)

            pltpu.make_async_copy(u_hbm.at[src_v.at[j + 1]], rows_b, gsb).wait()
            pltpu.async_copy(rows_b, acc_sh.at[dst_v.at[j + 1]], ssb, add=True)

            pltpu.make_async_copy(rows_a, acc_sh.at[dst_v.at[j]], ssa).wait()
            pltpu.async_copy(u_hbm.at[src_v.at[j + 2]], rows_a, gsa)

            pltpu.make_async_copy(rows_b, acc_sh.at[dst_v.at[j + 1]], ssb).wait()
            pltpu.async_copy(u_hbm.at[src_v.at[j + 3]], rows_b, gsb)

        pltpu.make_async_copy(u_hbm.at[src_v.at[GSZ - 2]], rows_a, gsa).wait()
        pltpu.sync_copy(rows_a, acc_sh.at[dst_v.at[GSZ - 2]], add=True)
        pltpu.make_async_copy(u_hbm.at[src_v.at[GSZ - 1]], rows_b, gsb).wait()
        pltpu.sync_copy(rows_b, acc_sh.at[dst_v.at[GSZ - 1]], add=True)

    plsc.subcore_barrier()

    @pl.loop(0, RPT - (RPT % CH), step=CH)
    def _(r):
        pltpu.sync_copy(acc_sh.at[pl.ds(s * RPT + r, CH)], rows_a)
        pltpu.sync_copy(rows_a, out_hbm.at[c, pl.ds(s * RPT + r, CH)])

    if RPT % CH:
        rem2 = RPT % CH
        pltpu.sync_copy(acc_sh.at[pl.ds(s * RPT + (RPT - rem2), rem2)],
                        rows_a.at[pl.ds(0, rem2)])
        pltpu.sync_copy(rows_a.at[pl.ds(0, rem2)],
                        out_hbm.at[c, pl.ds(s * RPT + (RPT - rem2), rem2)])


_PREC = lax.Precision.HIGHEST
_CP = pltpu.CompilerParams(vmem_limit_bytes=100 * 1024 * 1024)


def _tc_mm_body(x_ref, w_ref, o_ref):
    o_ref[...] = jnp.dot(x_ref[...], w_ref[...],
                         preferred_element_type=jnp.float32, precision=_PREC)


def _tc_mm(x, w):
    return pl.pallas_call(
        _tc_mm_body,
        out_shape=jax.ShapeDtypeStruct((x.shape[0], w.shape[1]), jnp.float32),
        compiler_params=_CP,
    )(x, w)


def _tc_scale_body(p_ref, dego_ref, u_ref, dinv_ref):
    deg = dego_ref[0, :N] + dego_ref[1, :N] + 1.0
    dinv = lax.rsqrt(deg)
    dinv_ref[...] = dinv
    u_ref[...] = p_ref[...] * dinv[:, None]


def _tc_scale(p, dego):
    return pl.pallas_call(
        _tc_scale_body,
        out_shape=(jax.ShapeDtypeStruct((N, H), jnp.float32),
                   jax.ShapeDtypeStruct((N,), jnp.float32)),
        compiler_params=_CP,
    )(p, dego)


def _bn_relu(y, g, be):
    mean = jnp.mean(y, axis=0)
    var = jnp.mean((y - mean) ** 2, axis=0)
    return jnp.maximum(g * (y - mean) / jnp.sqrt(var + EPS) + be, 0.0)


def _tc_mid_body(acc_ref, u_ref, dinv_ref, b_ref, g_ref, be_ref, w_ref,
                 un_ref):
    dinv = dinv_ref[...]
    y = (acc_ref[0, :N] + acc_ref[1, :N] + u_ref[...]) * dinv[:, None] + b_ref[...]
    h = _bn_relu(y, g_ref[...], be_ref[...])
    un_ref[...] = jnp.dot(h, w_ref[...], preferred_element_type=jnp.float32,
                          precision=_PREC) * dinv[:, None]


def _tc_mid(acc, u, dinv, b, g, be, w):
    return pl.pallas_call(
        _tc_mid_body,
        out_shape=jax.ShapeDtypeStruct((N, H), jnp.float32),
        compiler_params=_CP,
    )(acc, u, dinv, b, g, be, w)


def _tc_final_body(acc_ref, u_ref, dinv_ref, b_ref, g_ref, be_ref,
                   batch_ref, fcw_ref, fcb_ref, o_ref):
    dinv = dinv_ref[...]
    y = (acc_ref[0, :N] + acc_ref[1, :N] + u_ref[...]) * dinv[:, None] + b_ref[...]
    h = _bn_relu(y, g_ref[...], be_ref[...])
    gid = lax.broadcasted_iota(jnp.int32, (G, N), 0)
    onehot = (gid == batch_ref[...][None, :]).astype(jnp.float32)
    sums = jnp.dot(onehot, h, preferred_element_type=jnp.float32,
                   precision=_PREC)
    cnt = jnp.sum(onehot, axis=1)
    pooled = sums / jnp.maximum(cnt, 1.0)[:, None]
    out = jnp.dot(pooled, fcw_ref[...], preferred_element_type=jnp.float32,
                  precision=_PREC) + fcb_ref[...]
    o_ref[...] = jnp.maximum(out, 0.0)


def _tc_final(acc, u, dinv, b, g, be, batch, fcw, fcb):
    return pl.pallas_call(
        _tc_final_body,
        out_shape=jax.ShapeDtypeStruct((G, 128), jnp.float32),
        compiler_params=_CP,
    )(acc, u, dinv, b, g, be, batch, fcw, fcb)


def kernel(x, edge_index, batch, W1, b1, W2, b2, W3, b3,
           g1, be1, g2, be2, g3, be3, fcW, fcb):
    pad = E_PAD - E
    pad_iota = jnp.arange(pad, dtype=jnp.int32)
    pad_src = (pad_iota * 127) % N
    pad_dst = N + pad_iota % (NP - N)
    src_p = jnp.concatenate([edge_index[0], pad_src]).reshape(NW, K, CH)
    dst_p = jnp.concatenate([edge_index[1], pad_dst]).reshape(NW, K, CH)

    _sc_degree, _sc_edge_pass = _sc_kernels()

    dego = _sc_degree(dst_p).reshape(NC, NP)
    p1 = _tc_mm(x, W1)
    u1, dinv = _tc_scale(p1, dego)

    acc1 = _sc_edge_pass(u1, src_p, dst_p)
    u2 = _tc_mid(acc1, u1, dinv, b1, g1, be1, W2)
    acc2 = _sc_edge_pass(u2, src_p, dst_p)
    u3 = _tc_mid(acc2, u2, dinv, b2, g2, be2, W3)
    acc3 = _sc_edge_pass(u3, src_p, dst_p)
    return _tc_final(acc3, u3, dinv, b3, g3, be3, batch, fcW, fcb)


# speedup vs baseline: 1.1899x; 1.1899x over previous
import functools

import jax
import jax.numpy as jnp
import numpy as np
from jax import lax
from jax.experimental import pallas as pl
from jax.experimental.pallas import tpu as pltpu
from jax.experimental.pallas import tpu_sc as plsc

N = 10000
E = 320000
D = 128
H = 128
G = 64
EPS = 1e-5

NC = 2
NS = 16
NW = NC * NS
CH = 128
GSZ = 16
K = -(-E // (NW * CH * GSZ)) * GSZ
NG = K // GSZ
E_PAD = NW * K * CH
NP = 10112
RPT = NP // NS


@functools.cache
def _sc_kernels():
    mesh = plsc.VectorSubcoreMesh(core_axis_name="c", subcore_axis_name="s")
    deg = functools.partial(
        pl.kernel,
        out_type=jax.ShapeDtypeStruct((NC * NP,), jnp.float32),
        mesh=mesh,
        scratch_types=[
            pltpu.VMEM((K, CH), jnp.int32),
            pltpu.VMEM((CH,), jnp.float32),
            pltpu.VMEM((RPT,), jnp.float32),
            pltpu.VMEM_SHARED((NP,), jnp.float32),
        ],
    )(_sc_degree_body)
    edge = functools.partial(
        pl.kernel,
        out_type=jax.ShapeDtypeStruct((NC, NP, H), jnp.float32),
        mesh=mesh,
        scratch_types=[
            pltpu.VMEM((GSZ, CH), jnp.int32),
            pltpu.VMEM((GSZ, CH), jnp.int32),
            pltpu.VMEM((CH, H), jnp.float32),
            pltpu.VMEM((CH, H), jnp.float32),
            pltpu.VMEM_SHARED((NP, H), jnp.float32),
            pltpu.SemaphoreType.DMA,
            pltpu.SemaphoreType.DMA,
        ],
    )(_sc_edge_pass_body)
    return deg, edge


def _sc_degree_body(dst_hbm, out_hbm, dst_v, ones_v, stage_v, deg_sh):
    c = lax.axis_index("c")
    s = lax.axis_index("s")
    wid = c * NS + s

    @pl.loop(0, CH, step=16)
    def _(i):
        ones_v[pl.ds(i, 16)] = jnp.zeros((16,), jnp.float32)

    @pl.loop(0, RPT - (RPT % CH), step=CH)
    def _(r):
        pltpu.sync_copy(ones_v, deg_sh.at[pl.ds(s * RPT + r, CH)])

    rem = RPT % CH
    if rem:
        pltpu.sync_copy(ones_v.at[pl.ds(0, rem)],
                        deg_sh.at[pl.ds(s * RPT + (RPT - rem), rem)])

    @pl.loop(0, CH, step=16)
    def _(i):
        ones_v[pl.ds(i, 16)] = jnp.ones((16,), jnp.float32)

    pltpu.sync_copy(dst_hbm.at[wid], dst_v)
    plsc.subcore_barrier()

    @pl.loop(0, K)
    def _(j):
        pltpu.sync_copy(ones_v, deg_sh.at[dst_v.at[j]], add=True)

    plsc.subcore_barrier()
    pltpu.sync_copy(deg_sh.at[pl.ds(s * RPT, RPT)], stage_v)
    pltpu.sync_copy(stage_v, out_hbm.at[pl.ds(c * NP + s * RPT, RPT)])


def _sc_edge_pass_body(u_hbm, src_hbm, dst_hbm, out_hbm,
                       src_v, dst_v, rows_a, rows_b, acc_sh, gsa, gsb):
    c = lax.axis_index("c")
    s = lax.axis_index("s")
    wid = c * NS + s

    @pl.loop(0, CH)
    def _(i):
        @pl.loop(0, H, step=16)
        def _(j):
            rows_a[i, pl.ds(j, 16)] = jnp.zeros((16,), jnp.float32)

    @pl.loop(0, RPT - (RPT % CH), step=CH)
    def _(r):
        pltpu.sync_copy(rows_a, acc_sh.at[pl.ds(s * RPT + r, CH)])

    rem = RPT % CH
    if rem:
        pltpu.sync_copy(rows_a.at[pl.ds(0, rem)],
                        acc_sh.at[pl.ds(s * RPT + (RPT - rem), rem)])

    plsc.subcore_barrier()

    @pl.loop(0, NG)
    def _(g):
        pltpu.sync_copy(src_hbm.at[wid, pl.ds(g * GSZ, GSZ)], src_v)
        pltpu.sync_copy(dst_hbm.at[wid, pl.ds(g * GSZ, GSZ)], dst_v)

        pltpu.async_copy(u_hbm.at[src_v.at[0]], rows_a, gsa)
        pltpu.async_copy(u_hbm.at[src_v.at[1]], rows_b, gsb)

        @pl.loop(0, GSZ - 2, step=2)
        def _(j):
            pltpu.make_async_copy(u_hbm.at[src_v.at[j]], rows_a, gsa).wait()
            pltpu.sync_copy(rows_a, acc_sh.at[dst_v.at[j]], add=True)
            pltpu.async_copy(u_hbm.at[src_v.at[j + 2]], rows_a, gsa)

            pltpu.make_async_copy(u_hbm.at[src_v.at[j + 1]], rows_b, gsb).wait()
            pltpu.sync_copy(rows_b, acc_sh.at[dst_v.at[j + 1]], add=True)
            pltpu.async_copy(u_hbm.at[src_v.at[j + 3]], rows_b, gsb)

        pltpu.make_async_copy(u_hbm.at[src_v.at[GSZ - 2]], rows_a, gsa).wait()
        pltpu.sync_copy(rows_a, acc_sh.at[dst_v.at[GSZ - 2]], add=True)
        pltpu.make_async_copy(u_hbm.at[src_v.at[GSZ - 1]], rows_b, gsb).wait()
        pltpu.sync_copy(rows_b, acc_sh.at[dst_v.at[GSZ - 1]], add=True)

    plsc.subcore_barrier()

    @pl.loop(0, RPT - (RPT % CH), step=CH)
    def _(r):
        pltpu.sync_copy(acc_sh.at[pl.ds(s * RPT + r, CH)], rows_a)
        pltpu.sync_copy(rows_a, out_hbm.at[c, pl.ds(s * RPT + r, CH)])

    if RPT % CH:
        rem2 = RPT % CH
        pltpu.sync_copy(acc_sh.at[pl.ds(s * RPT + (RPT - rem2), rem2)],
                        rows_a.at[pl.ds(0, rem2)])
        pltpu.sync_copy(rows_a.at[pl.ds(0, rem2)],
                        out_hbm.at[c, pl.ds(s * RPT + (RPT - rem2), rem2)])


_PREC = lax.Precision.HIGHEST
_CP = pltpu.CompilerParams(vmem_limit_bytes=100 * 1024 * 1024)


def _tc_mm_body(x_ref, w_ref, o_ref):
    o_ref[...] = jnp.dot(x_ref[...], w_ref[...],
                         preferred_element_type=jnp.float32, precision=_PREC)


def _tc_mm(x, w):
    return pl.pallas_call(
        _tc_mm_body,
        out_shape=jax.ShapeDtypeStruct((x.shape[0], w.shape[1]), jnp.float32),
        compiler_params=_CP,
    )(x, w)


def _tc_scale_body(p_ref, dego_ref, u_ref, dinv_ref):
    deg = dego_ref[0, :N] + dego_ref[1, :N] + 1.0
    dinv = lax.rsqrt(deg)
    dinv_ref[...] = dinv
    u_ref[...] = p_ref[...] * dinv[:, None]


def _tc_scale(p, dego):
    return pl.pallas_call(
        _tc_scale_body,
        out_shape=(jax.ShapeDtypeStruct((N, H), jnp.float32),
                   jax.ShapeDtypeStruct((N,), jnp.float32)),
        compiler_params=_CP,
    )(p, dego)


def _bn_relu(y, g, be):
    mean = jnp.mean(y, axis=0)
    var = jnp.mean((y - mean) ** 2, axis=0)
    return jnp.maximum(g * (y - mean) / jnp.sqrt(var + EPS) + be, 0.0)


def _tc_mid_body(acc_ref, u_ref, dinv_ref, b_ref, g_ref, be_ref, w_ref,
                 un_ref):
    dinv = dinv_ref[...]
    y = (acc_ref[0, :N] + acc_ref[1, :N] + u_ref[...]) * dinv[:, None] + b_ref[...]
    h = _bn_relu(y, g_ref[...], be_ref[...])
    un_ref[...] = jnp.dot(h, w_ref[...], preferred_element_type=jnp.float32,
                          precision=_PREC) * dinv[:, None]


def _tc_mid(acc, u, dinv, b, g, be, w):
    return pl.pallas_call(
        _tc_mid_body,
        out_shape=jax.ShapeDtypeStruct((N, H), jnp.float32),
        compiler_params=_CP,
    )(acc, u, dinv, b, g, be, w)


def _tc_final_body(acc_ref, u_ref, dinv_ref, b_ref, g_ref, be_ref,
                   batch_ref, fcw_ref, fcb_ref, o_ref):
    dinv = dinv_ref[...]
    y = (acc_ref[0, :N] + acc_ref[1, :N] + u_ref[...]) * dinv[:, None] + b_ref[...]
    h = _bn_relu(y, g_ref[...], be_ref[...])
    gid = lax.broadcasted_iota(jnp.int32, (G, N), 0)
    onehot = (gid == batch_ref[...][None, :]).astype(jnp.float32)
    sums = jnp.dot(onehot, h, preferred_element_type=jnp.float32,
                   precision=_PREC)
    cnt = jnp.sum(onehot, axis=1)
    pooled = sums / jnp.maximum(cnt, 1.0)[:, None]
    out = jnp.dot(pooled, fcw_ref[...], preferred_element_type=jnp.float32,
                  precision=_PREC) + fcb_ref[...]
    o_ref[...] = jnp.maximum(out, 0.0)


def _tc_final(acc, u, dinv, b, g, be, batch, fcw, fcb):
    return pl.pallas_call(
        _tc_final_body,
        out_shape=jax.ShapeDtypeStruct((G, 128), jnp.float32),
        compiler_params=_CP,
    )(acc, u, dinv, b, g, be, batch, fcw, fcb)


def kernel(x, edge_index, batch, W1, b1, W2, b2, W3, b3,
           g1, be1, g2, be2, g3, be3, fcW, fcb):
    pad = E_PAD - E
    pad_iota = np.arange(pad, dtype=np.int32)
    pad_src = jnp.asarray((pad_iota * 127) % N)
    pad_dst = jnp.asarray(N + pad_iota % (NP - N))
    src_p = jnp.concatenate([edge_index[0], pad_src]).reshape(NW, K, CH)
    dst_p = jnp.concatenate([edge_index[1], pad_dst]).reshape(NW, K, CH)

    _sc_degree, _sc_edge_pass = _sc_kernels()

    dego = _sc_degree(dst_p).reshape(NC, NP)
    p1 = _tc_mm(x, W1)
    u1, dinv = _tc_scale(p1, dego)

    acc1 = _sc_edge_pass(u1, src_p, dst_p)
    u2 = _tc_mid(acc1, u1, dinv, b1, g1, be1, W2)
    acc2 = _sc_edge_pass(u2, src_p, dst_p)
    u3 = _tc_mid(acc2, u2, dinv, b2, g2, be2, W3)
    acc3 = _sc_edge_pass(u3, src_p, dst_p)
    return _tc_final(acc3, u3, dinv, b3, g3, be3, batch, fcW, fcb)
